# Initial kernel scaffold; baseline (speedup 1.0000x reference)
import jax
import jax.numpy as jnp
from jax.experimental import pallas as pl


def kernel(x, edge_index, W):
    raise NotImplementedError("write your pallas kernel here")


# speedup vs baseline: 8.1137x; 8.1137x over previous
import functools

import jax
import jax.numpy as jnp
from jax import lax
from jax.experimental import pallas as pl
from jax.experimental.pallas import tpu as pltpu
from jax.experimental.pallas import tpu_sc as plsc


def _mm_body(x_ref, w_ref, o_ref):
    o_ref[...] = jnp.maximum(
        jnp.dot(x_ref[...], w_ref[...], preferred_element_type=jnp.float32), 0.0)


def _relu_matmul(x, W):
    N, Din = x.shape
    Dout = W.shape[1]
    BN = 1000
    grid = (N // BN,)
    return pl.pallas_call(
        _mm_body,
        grid=grid,
        in_specs=[
            pl.BlockSpec((BN, Din), lambda i: (i, 0)),
            pl.BlockSpec((Din, Dout), lambda i: (0, 0)),
        ],
        out_specs=pl.BlockSpec((BN, Dout), lambda i: (i, 0)),
        out_shape=jax.ShapeDtypeStruct((N, Dout), jnp.float32),
    )(x, W)


def _make_sc_aggregate(N, E, D, Npad, C, ept, rpt):
    info = plsc.get_sparse_core_info()
    NC, NS = info.num_cores, info.num_subcores
    nchunk = ept // C
    ZR = rpt // 8

    mesh = plsc.VectorSubcoreMesh(core_axis_name="c", subcore_axis_name="s")

    @functools.partial(
        pl.kernel,
        out_type=[
            jax.ShapeDtypeStruct((NC, Npad, D), jnp.float32),
            jax.ShapeDtypeStruct((NC, Npad, 16), jnp.float32),
        ],
        mesh=mesh,
        compiler_params=pltpu.CompilerParams(use_tc_tiling_on_sc=False),
        scratch_types=[
            pltpu.VMEM_SHARED((Npad, D), jnp.float32),
            pltpu.VMEM_SHARED((Npad, 16), jnp.float32),
            pltpu.VMEM((C,), jnp.int32),
            pltpu.VMEM((C,), jnp.int32),
            pltpu.VMEM((C,), jnp.int32),
            pltpu.VMEM((C, D), jnp.float32),
            pltpu.VMEM((C, 16), jnp.float32),
            pltpu.VMEM((ZR, D), jnp.float32),
            pltpu.VMEM((ZR, 16), jnp.float32),
            pltpu.SemaphoreType.DMA,
        ],
    )
    def sc_agg(feat_hbm, edge_hbm, sum_hbm, cnt_hbm,
               acc, cnt, row_v, col_v, rowm_v, rows_v, ones_v, z_v, zc_v, sem):
        c = lax.axis_index("c")
        s = lax.axis_index("s")
        wid = c * NS + s

        zero16 = jnp.zeros((16,), jnp.float32)
        one16 = jnp.ones((16,), jnp.float32)

        def fill_ones(i, _):
            ones_v[i, :] = one16
            return 0
        lax.fori_loop(0, C, fill_ones, 0)

        def fill_z(i, _):
            z_v[i // (D // 16), pl.ds((i % (D // 16)) * 16, 16)] = zero16
            return 0
        lax.fori_loop(0, ZR * (D // 16), fill_z, 0)

        def fill_zc(i, _):
            zc_v[i, :] = zero16
            return 0
        lax.fori_loop(0, ZR, fill_zc, 0)

        rbase = s * rpt
        for k in range(8):
            pltpu.sync_copy(z_v, acc.at[pl.ds(rbase + k * ZR, ZR)])
            pltpu.sync_copy(zc_v, cnt.at[pl.ds(rbase + k * ZR, ZR)])
        plsc.subcore_barrier()

        ebase = wid * ept

        def chunk(i, _):
            off = ebase + i * C
            pltpu.sync_copy(edge_hbm.at[pl.ds(off, C)], row_v)
            pltpu.sync_copy(edge_hbm.at[pl.ds(E + off, C)], col_v)
            for k in range(C // 16):
                r = row_v[pl.ds(k * 16, 16)]
                cc = col_v[pl.ds(k * 16, 16)]
                rowm_v[pl.ds(k * 16, 16)] = jnp.where(r == cc, N, r)
            pltpu.async_copy(feat_hbm.at[col_v], rows_v, sem).wait()
            pltpu.sync_copy(rows_v, acc.at[rowm_v], add=True)
            pltpu.sync_copy(ones_v, cnt.at[rowm_v], add=True)
            return 0
        lax.fori_loop(0, nchunk, chunk, 0)

        plsc.subcore_barrier()
        pltpu.sync_copy(acc.at[pl.ds(rbase, rpt)], sum_hbm.at[c, pl.ds(rbase, rpt)])
        pltpu.sync_copy(cnt.at[pl.ds(rbase, rpt)], cnt_hbm.at[c, pl.ds(rbase, rpt)])

    return sc_agg


def _div_body(ps_ref, pc_ref, feat_ref, o_ref):
    total = ps_ref[0] + ps_ref[1] + feat_ref[...]
    den = pc_ref[0, :, 0:1] + pc_ref[1, :, 0:1] + 1.0
    o_ref[...] = total / den


def _combine(psum, pcnt, feat):
    N, D = feat.shape
    BN = 400
    grid = (N // BN,)
    return pl.pallas_call(
        _div_body,
        grid=grid,
        in_specs=[
            pl.BlockSpec((2, BN, D), lambda i: (0, i, 0)),
            pl.BlockSpec((2, BN, 16), lambda i: (0, i, 0)),
            pl.BlockSpec((BN, D), lambda i: (i, 0)),
        ],
        out_specs=pl.BlockSpec((BN, D), lambda i: (i, 0)),
        out_shape=jax.ShapeDtypeStruct((N, D), jnp.float32),
    )(psum, pcnt, feat)


def kernel(x, edge_index, W):
    N, _ = x.shape
    D = W.shape[1]
    E = edge_index.shape[1]

    info = plsc.get_sparse_core_info()
    NW = info.num_cores * info.num_subcores
    ept = E // NW
    C = next(c for c in (128, 120, 112, 104, 96, 88, 80, 72, 64, 56, 48, 40,
                         32, 24, 16, 8) if ept % c == 0)
    rpt = -(-(N + 1) // info.num_subcores)
    rpt = -(-rpt // 8) * 8
    Npad = rpt * info.num_subcores

    feat = _relu_matmul(x, W)
    psum, pcnt = _make_sc_aggregate(N, E, D, Npad, C, ept, rpt)(
        feat, edge_index.reshape(-1))
    return _combine(psum, pcnt, feat)


# speedup vs baseline: 12.2379x; 1.5083x over previous
import functools

import jax
import jax.numpy as jnp
from jax import lax
from jax.experimental import pallas as pl
from jax.experimental.pallas import tpu as pltpu
from jax.experimental.pallas import tpu_sc as plsc


def _mm_body(x_ref, w_ref, o_ref):
    o_ref[...] = jnp.maximum(
        jnp.dot(x_ref[...], w_ref[...], preferred_element_type=jnp.float32), 0.0)


def _relu_matmul(x, W):
    N, Din = x.shape
    Dout = W.shape[1]
    BN = 1000
    grid = (N // BN,)
    return pl.pallas_call(
        _mm_body,
        grid=grid,
        in_specs=[
            pl.BlockSpec((BN, Din), lambda i: (i, 0)),
            pl.BlockSpec((Din, Dout), lambda i: (0, 0)),
        ],
        out_specs=pl.BlockSpec((BN, Dout), lambda i: (i, 0)),
        out_shape=jax.ShapeDtypeStruct((N, Dout), jnp.float32),
    )(x, W)


def _make_sc_aggregate(N, E, D, Npad, C, ept, rpt):
    info = plsc.get_sparse_core_info()
    NC, NS = info.num_cores, info.num_subcores
    nchunk = ept // C
    ZR = rpt // 8

    mesh = plsc.VectorSubcoreMesh(core_axis_name="c", subcore_axis_name="s")

    @functools.partial(
        pl.kernel,
        out_type=[
            jax.ShapeDtypeStruct((NC, Npad, D), jnp.float32),
            jax.ShapeDtypeStruct((NC, Npad, 16), jnp.float32),
        ],
        mesh=mesh,
        compiler_params=pltpu.CompilerParams(use_tc_tiling_on_sc=False),
        scratch_types=[
            pltpu.VMEM_SHARED((Npad, D), jnp.float32),
            pltpu.VMEM_SHARED((Npad, 16), jnp.float32),
            pltpu.VMEM((2, C), jnp.int32),
            pltpu.VMEM((2, C), jnp.int32),
            pltpu.VMEM((2, C), jnp.int32),
            pltpu.VMEM((2, C, D), jnp.float32),
            pltpu.VMEM((C, 16), jnp.float32),
            pltpu.VMEM((ZR, D), jnp.float32),
            pltpu.VMEM((ZR, 16), jnp.float32),
            pltpu.SemaphoreType.DMA((2,)),
        ],
    )
    def sc_agg(feat_hbm, edge_hbm, sum_hbm, cnt_hbm,
               acc, cnt, row_v, col_v, rowm_v, rows_v, ones_v, z_v, zc_v, sem):
        c = lax.axis_index("c")
        s = lax.axis_index("s")
        wid = c * NS + s

        zero16 = jnp.zeros((16,), jnp.float32)
        one16 = jnp.ones((16,), jnp.float32)

        def fill_ones(i, _):
            ones_v[i, :] = one16
            return 0
        lax.fori_loop(0, C, fill_ones, 0)

        def fill_z(i, _):
            z_v[i // (D // 16), pl.ds((i % (D // 16)) * 16, 16)] = zero16
            return 0
        lax.fori_loop(0, ZR * (D // 16), fill_z, 0)

        def fill_zc(i, _):
            zc_v[i, :] = zero16
            return 0
        lax.fori_loop(0, ZR, fill_zc, 0)

        rbase = s * rpt
        for k in range(8):
            pltpu.sync_copy(z_v, acc.at[pl.ds(rbase + k * ZR, ZR)])
            pltpu.sync_copy(zc_v, cnt.at[pl.ds(rbase + k * ZR, ZR)])
        plsc.subcore_barrier()

        ebase = wid * ept

        def stage(i):
            b = lax.rem(i, 2)
            off = ebase + i * C
            pltpu.sync_copy(edge_hbm.at[pl.ds(off, C)], row_v.at[b])
            pltpu.sync_copy(edge_hbm.at[pl.ds(E + off, C)], col_v.at[b])
            for k in range(C // 16):
                r = row_v[b, pl.ds(k * 16, 16)]
                cc = col_v[b, pl.ds(k * 16, 16)]
                rowm_v[b, pl.ds(k * 16, 16)] = jnp.where(r == cc, N, r)
            pltpu.async_copy(feat_hbm.at[col_v.at[b]], rows_v.at[b], sem.at[b])

        def drain(i):
            b = lax.rem(i, 2)
            pltpu.make_async_copy(
                feat_hbm.at[col_v.at[b]], rows_v.at[b], sem.at[b]).wait()
            pltpu.sync_copy(rows_v.at[b], acc.at[rowm_v.at[b]], add=True)
            pltpu.sync_copy(ones_v, cnt.at[rowm_v.at[b]], add=True)

        stage(0)

        def chunk(i, _):
            @pl.when(i + 1 < nchunk)
            def _():
                stage(i + 1)
            drain(i)
            return 0
        lax.fori_loop(0, nchunk, chunk, 0)

        plsc.subcore_barrier()
        pltpu.sync_copy(acc.at[pl.ds(rbase, rpt)], sum_hbm.at[c, pl.ds(rbase, rpt)])
        pltpu.sync_copy(cnt.at[pl.ds(rbase, rpt)], cnt_hbm.at[c, pl.ds(rbase, rpt)])

    return sc_agg


def _div_body(ps_ref, pc_ref, feat_ref, o_ref):
    total = ps_ref[0] + ps_ref[1] + feat_ref[...]
    den = pc_ref[0, :, 0:1] + pc_ref[1, :, 0:1] + 1.0
    o_ref[...] = total / den


def _combine(psum, pcnt, feat):
    N, D = feat.shape
    BN = 400
    grid = (N // BN,)
    return pl.pallas_call(
        _div_body,
        grid=grid,
        in_specs=[
            pl.BlockSpec((2, BN, D), lambda i: (0, i, 0)),
            pl.BlockSpec((2, BN, 16), lambda i: (0, i, 0)),
            pl.BlockSpec((BN, D), lambda i: (i, 0)),
        ],
        out_specs=pl.BlockSpec((BN, D), lambda i: (i, 0)),
        out_shape=jax.ShapeDtypeStruct((N, D), jnp.float32),
    )(psum, pcnt, feat)


def kernel(x, edge_index, W):
    N, _ = x.shape
    D = W.shape[1]
    E = edge_index.shape[1]

    info = plsc.get_sparse_core_info()
    NW = info.num_cores * info.num_subcores
    ept = E // NW
    C = next(c for c in (128, 120, 112, 104, 96, 88, 80, 72, 64, 56, 48, 40,
                         32, 24, 16, 8) if ept % c == 0)
    rpt = -(-(N + 1) // info.num_subcores)
    rpt = -(-rpt // 8) * 8
    Npad = rpt * info.num_subcores

    feat = _relu_matmul(x, W)
    psum, pcnt = _make_sc_aggregate(N, E, D, Npad, C, ept, rpt)(
        feat, edge_index.reshape(-1))
    return _combine(psum, pcnt, feat)


# speedup vs baseline: 14.2341x; 1.1631x over previous
import functools

import jax
import jax.numpy as jnp
from jax import lax
from jax.experimental import pallas as pl
from jax.experimental.pallas import tpu as pltpu
from jax.experimental.pallas import tpu_sc as plsc


def _mm_body(x_ref, w_ref, o_ref):
    o_ref[...] = jnp.maximum(
        jnp.dot(x_ref[...], w_ref[...], preferred_element_type=jnp.float32), 0.0)


def _relu_matmul(x, W):
    N, Din = x.shape
    Dout = W.shape[1]
    BN = 1000
    grid = (N // BN,)
    return pl.pallas_call(
        _mm_body,
        grid=grid,
        in_specs=[
            pl.BlockSpec((BN, Din), lambda i: (i, 0)),
            pl.BlockSpec((Din, Dout), lambda i: (0, 0)),
        ],
        out_specs=pl.BlockSpec((BN, Dout), lambda i: (i, 0)),
        out_shape=jax.ShapeDtypeStruct((N, Dout), jnp.float32),
    )(x, W)


def _make_sc_aggregate(N, E, D, Npad, C, rpt):
    info = plsc.get_sparse_core_info()
    NC, NS = info.num_cores, info.num_subcores
    NW = NC * NS
    nch = E // C
    base_ch = nch // NW
    extra = nch - base_ch * NW
    ZR = rpt // 8

    mesh = plsc.VectorSubcoreMesh(core_axis_name="c", subcore_axis_name="s")

    @functools.partial(
        pl.kernel,
        out_type=[
            jax.ShapeDtypeStruct((NC, Npad, D), jnp.float32),
            jax.ShapeDtypeStruct((NC, Npad, 16), jnp.float32),
        ],
        mesh=mesh,
        compiler_params=pltpu.CompilerParams(use_tc_tiling_on_sc=False),
        scratch_types=[
            pltpu.VMEM_SHARED((Npad, D), jnp.float32),
            pltpu.VMEM_SHARED((Npad, 16), jnp.float32),
            pltpu.VMEM((2, C), jnp.int32),
            pltpu.VMEM((2, C), jnp.int32),
            pltpu.VMEM((2, C), jnp.int32),
            pltpu.VMEM((2, C, D), jnp.float32),
            pltpu.VMEM((C, 16), jnp.float32),
            pltpu.VMEM((ZR, 16), jnp.float32),
            pltpu.SemaphoreType.DMA((2,)),
        ],
    )
    def sc_agg(feat_hbm, edge_hbm, sum_hbm, cnt_hbm,
               acc, cnt, row_v, col_v, rowm_v, rows_v, ones_v, zc_v, sem):
        c = lax.axis_index("c")
        s = lax.axis_index("s")
        wid = c * NS + s

        zero16 = jnp.zeros((16,), jnp.float32)
        one16 = jnp.ones((16,), jnp.float32)

        def fill_ones(i, _):
            ones_v[i, :] = one16
            return 0
        lax.fori_loop(0, C, fill_ones, 0)

        def fill_zc(i, _):
            zc_v[i, :] = zero16
            return 0
        lax.fori_loop(0, ZR, fill_zc, 0)

        def fill_z(i, _):
            rows_v[0, i // (D // 16), pl.ds((i % (D // 16)) * 16, 16)] = zero16
            return 0
        lax.fori_loop(0, C * (D // 16), fill_z, 0)

        rbase = s * rpt
        nz = rpt // C
        rem = rpt - nz * C
        for k in range(nz):
            pltpu.sync_copy(rows_v.at[0], acc.at[pl.ds(rbase + k * C, C)])
        if rem:
            pltpu.sync_copy(rows_v.at[0, pl.ds(0, rem)],
                            acc.at[pl.ds(rbase + nz * C, rem)])
        for k in range(8):
            pltpu.sync_copy(zc_v, cnt.at[pl.ds(rbase + k * ZR, ZR)])
        plsc.subcore_barrier()

        cstart = base_ch * wid + jnp.minimum(wid, extra)
        ncw = base_ch + jnp.where(wid < extra, 1, 0)

        def stage(i):
            b = lax.rem(i, 2)
            off = (cstart + i) * C
            pltpu.sync_copy(edge_hbm.at[pl.ds(off, C)], row_v.at[b])
            pltpu.sync_copy(edge_hbm.at[pl.ds(E + off, C)], col_v.at[b])
            for k in range(C // 16):
                r = row_v[b, pl.ds(k * 16, 16)]
                cc = col_v[b, pl.ds(k * 16, 16)]
                rowm_v[b, pl.ds(k * 16, 16)] = jnp.where(r == cc, N, r)
            pltpu.async_copy(feat_hbm.at[col_v.at[b]], rows_v.at[b], sem.at[b])

        def drain(i):
            b = lax.rem(i, 2)
            pltpu.make_async_copy(
                feat_hbm.at[col_v.at[b]], rows_v.at[b], sem.at[b]).wait()
            pltpu.sync_copy(rows_v.at[b], acc.at[rowm_v.at[b]], add=True)
            pltpu.sync_copy(ones_v, cnt.at[rowm_v.at[b]], add=True)

        stage(0)

        def chunk(i, _):
            @pl.when(i + 1 < ncw)
            def _():
                stage(i + 1)
            drain(i)
            return 0
        lax.fori_loop(0, ncw, chunk, 0)

        plsc.subcore_barrier()
        pltpu.sync_copy(acc.at[pl.ds(rbase, rpt)], sum_hbm.at[c, pl.ds(rbase, rpt)])
        pltpu.sync_copy(cnt.at[pl.ds(rbase, rpt)], cnt_hbm.at[c, pl.ds(rbase, rpt)])

    return sc_agg


def _div_body(ps_ref, pc_ref, feat_ref, o_ref):
    total = ps_ref[0] + ps_ref[1] + feat_ref[...]
    den = pc_ref[0, :, 0:1] + pc_ref[1, :, 0:1] + 1.0
    o_ref[...] = total / den


def _combine(psum, pcnt, feat):
    N, D = feat.shape
    BN = 400
    grid = (N // BN,)
    return pl.pallas_call(
        _div_body,
        grid=grid,
        in_specs=[
            pl.BlockSpec((2, BN, D), lambda i: (0, i, 0)),
            pl.BlockSpec((2, BN, 16), lambda i: (0, i, 0)),
            pl.BlockSpec((BN, D), lambda i: (i, 0)),
        ],
        out_specs=pl.BlockSpec((BN, D), lambda i: (i, 0)),
        out_shape=jax.ShapeDtypeStruct((N, D), jnp.float32),
    )(psum, pcnt, feat)


def kernel(x, edge_index, W):
    N, _ = x.shape
    D = W.shape[1]
    E = edge_index.shape[1]

    info = plsc.get_sparse_core_info()
    NS = info.num_subcores
    C = 128
    assert E % C == 0
    rpt = -(-(N + 1) // NS)
    rpt = -(-rpt // 8) * 8
    Npad = rpt * NS

    feat = _relu_matmul(x, W)
    psum, pcnt = _make_sc_aggregate(N, E, D, Npad, C, rpt)(
        feat, edge_index.reshape(-1))
    return _combine(psum, pcnt, feat)


# speedup vs baseline: 16.1672x; 1.1358x over previous
import functools

import jax
import jax.numpy as jnp
from jax import lax
from jax.experimental import pallas as pl
from jax.experimental.pallas import tpu as pltpu
from jax.experimental.pallas import tpu_sc as plsc


def _mm_body(x_ref, w_ref, o_ref):
    o_ref[...] = jnp.maximum(
        jnp.dot(x_ref[...], w_ref[...], preferred_element_type=jnp.float32), 0.0)


def _relu_matmul(x, W):
    N, Din = x.shape
    Dout = W.shape[1]
    BN = 1000
    grid = (N // BN,)
    return pl.pallas_call(
        _mm_body,
        grid=grid,
        in_specs=[
            pl.BlockSpec((BN, Din), lambda i: (i, 0)),
            pl.BlockSpec((Din, Dout), lambda i: (0, 0)),
        ],
        out_specs=pl.BlockSpec((BN, Dout), lambda i: (i, 0)),
        out_shape=jax.ShapeDtypeStruct((N, Dout), jnp.float32),
    )(x, W)


def _make_sc_aggregate(N, E, D, Npad, C, rpt):
    info = plsc.get_sparse_core_info()
    NC, NS = info.num_cores, info.num_subcores
    NW = NC * NS
    nch = E // C
    base_ch = nch // NW
    extra = nch - base_ch * NW
    GK = 6
    ngr = base_ch // GK
    assert ngr * GK == base_ch
    ZR = rpt // 8

    mesh = plsc.VectorSubcoreMesh(core_axis_name="c", subcore_axis_name="s")

    @functools.partial(
        pl.kernel,
        out_type=[
            jax.ShapeDtypeStruct((NC, Npad, D), jnp.float32),
            jax.ShapeDtypeStruct((NC, Npad, 16), jnp.float32),
        ],
        mesh=mesh,
        compiler_params=pltpu.CompilerParams(use_tc_tiling_on_sc=False),
        scratch_types=[
            pltpu.VMEM_SHARED((Npad, D), jnp.float32),
            pltpu.VMEM_SHARED((Npad, 16), jnp.float32),
            pltpu.VMEM((2, GK * C), jnp.int32),
            pltpu.VMEM((2, GK * C), jnp.int32),
            pltpu.VMEM((2, C), jnp.int32),
            pltpu.VMEM((2, C, D), jnp.float32),
            pltpu.VMEM((C, 16), jnp.float32),
            pltpu.VMEM((ZR, 16), jnp.float32),
            pltpu.SemaphoreType.DMA((2,)),
            pltpu.SemaphoreType.DMA((2,)),
        ],
    )
    def sc_agg(feat_hbm, edge_hbm, sum_hbm, cnt_hbm,
               acc, cnt, grow_v, gcol_v, rowm_v, rows_v, ones_v, zc_v,
               sem, gsem):
        c = lax.axis_index("c")
        s = lax.axis_index("s")
        wid = c * NS + s

        zero16 = jnp.zeros((16,), jnp.float32)
        one16 = jnp.ones((16,), jnp.float32)

        def fill_ones(i, _):
            ones_v[i, :] = one16
            return 0
        lax.fori_loop(0, C, fill_ones, 0)

        def fill_zc(i, _):
            zc_v[i, :] = zero16
            return 0
        lax.fori_loop(0, ZR, fill_zc, 0)

        def fill_z(i, _):
            rows_v[0, i // (D // 16), pl.ds((i % (D // 16)) * 16, 16)] = zero16
            return 0
        lax.fori_loop(0, C * (D // 16), fill_z, 0)

        rbase = s * rpt
        nz = rpt // C
        rem_rows = rpt - nz * C
        for k in range(nz):
            pltpu.sync_copy(rows_v.at[0], acc.at[pl.ds(rbase + k * C, C)])
        if rem_rows:
            pltpu.sync_copy(rows_v.at[0, pl.ds(0, rem_rows)],
                            acc.at[pl.ds(rbase + nz * C, rem_rows)])
        for k in range(8):
            pltpu.sync_copy(zc_v, cnt.at[pl.ds(rbase + k * ZR, ZR)])
        plsc.subcore_barrier()

        cstart = base_ch * wid

        def load_group(g):
            gb = lax.rem(g, 2)
            off = (cstart + g * GK) * C
            pltpu.async_copy(edge_hbm.at[pl.ds(off, GK * C)],
                             grow_v.at[gb], gsem.at[gb])
            pltpu.async_copy(edge_hbm.at[pl.ds(E + off, GK * C)],
                             gcol_v.at[gb], gsem.at[gb])

        def wait_group(g):
            gb = lax.rem(g, 2)
            pltpu.make_async_copy(edge_hbm.at[pl.ds(0, GK * C)],
                                  grow_v.at[gb], gsem.at[gb]).wait()
            pltpu.make_async_copy(edge_hbm.at[pl.ds(0, GK * C)],
                                  gcol_v.at[gb], gsem.at[gb]).wait()

        def stage(i):
            b = lax.rem(i, 2)
            g = i // GK
            gb = lax.rem(g, 2)
            kofs = lax.rem(i, GK) * C
            for j in range(C // 16):
                r = grow_v[gb, pl.ds(kofs + j * 16, 16)]
                cc = gcol_v[gb, pl.ds(kofs + j * 16, 16)]
                rowm_v[b, pl.ds(j * 16, 16)] = jnp.where(r == cc, N, r)
            pltpu.async_copy(feat_hbm.at[gcol_v.at[gb, pl.ds(kofs, C)]],
                             rows_v.at[b], sem.at[b])

        def drain(i):
            b = lax.rem(i, 2)
            g = i // GK
            gb = lax.rem(g, 2)
            kofs = lax.rem(i, GK) * C
            pltpu.make_async_copy(
                feat_hbm.at[gcol_v.at[gb, pl.ds(kofs, C)]],
                rows_v.at[b], sem.at[b]).wait()
            pltpu.sync_copy(rows_v.at[b], acc.at[rowm_v.at[b]], add=True)
            pltpu.sync_copy(ones_v, cnt.at[rowm_v.at[b]], add=True)

        ntot = ngr * GK
        load_group(0)

        def body(i, _):
            g = i // GK
            at_group = lax.rem(i, GK) == 0

            @pl.when(jnp.logical_and(at_group, i > 0))
            def _():
                drain(i - 1)

            @pl.when(at_group)
            def _():
                wait_group(g)

            @pl.when(jnp.logical_and(at_group, g + 1 < ngr))
            def _():
                load_group(g + 1)

            stage(i)

            @pl.when(jnp.logical_and(jnp.logical_not(at_group), i > 0))
            def _():
                drain(i - 1)
            return 0
        lax.fori_loop(0, ntot, body, 0)
        drain(ntot - 1)

        @pl.when(wid < extra)
        def _():
            toff = (NW * base_ch + wid) * C
            pltpu.sync_copy(edge_hbm.at[pl.ds(toff, C)],
                            grow_v.at[0, pl.ds(0, C)])
            pltpu.sync_copy(edge_hbm.at[pl.ds(E + toff, C)],
                            gcol_v.at[0, pl.ds(0, C)])
            for j in range(C // 16):
                r = grow_v[0, pl.ds(j * 16, 16)]
                cc = gcol_v[0, pl.ds(j * 16, 16)]
                rowm_v[0, pl.ds(j * 16, 16)] = jnp.where(r == cc, N, r)
            pltpu.async_copy(feat_hbm.at[gcol_v.at[0, pl.ds(0, C)]],
                             rows_v.at[0], sem.at[0])
            pltpu.make_async_copy(
                feat_hbm.at[gcol_v.at[0, pl.ds(0, C)]],
                rows_v.at[0], sem.at[0]).wait()
            pltpu.sync_copy(rows_v.at[0], acc.at[rowm_v.at[0]], add=True)
            pltpu.sync_copy(ones_v, cnt.at[rowm_v.at[0]], add=True)

        plsc.subcore_barrier()
        pltpu.sync_copy(acc.at[pl.ds(rbase, rpt)], sum_hbm.at[c, pl.ds(rbase, rpt)])
        pltpu.sync_copy(cnt.at[pl.ds(rbase, rpt)], cnt_hbm.at[c, pl.ds(rbase, rpt)])

    return sc_agg


def _div_body(ps_ref, pc_ref, feat_ref, o_ref):
    total = ps_ref[0] + ps_ref[1] + feat_ref[...]
    den = pc_ref[0, :, 0:1] + pc_ref[1, :, 0:1] + 1.0
    o_ref[...] = total / den


def _combine(psum, pcnt, feat):
    N, D = feat.shape
    BN = 400
    grid = (N // BN,)
    return pl.pallas_call(
        _div_body,
        grid=grid,
        in_specs=[
            pl.BlockSpec((2, BN, D), lambda i: (0, i, 0)),
            pl.BlockSpec((2, BN, 16), lambda i: (0, i, 0)),
            pl.BlockSpec((BN, D), lambda i: (i, 0)),
        ],
        out_specs=pl.BlockSpec((BN, D), lambda i: (i, 0)),
        out_shape=jax.ShapeDtypeStruct((N, D), jnp.float32),
    )(psum, pcnt, feat)


def kernel(x, edge_index, W):
    N, _ = x.shape
    D = W.shape[1]
    E = edge_index.shape[1]

    info = plsc.get_sparse_core_info()
    NS = info.num_subcores
    C = 128
    assert E % C == 0
    rpt = -(-(N + 1) // NS)
    rpt = -(-rpt // 8) * 8
    Npad = rpt * NS

    feat = _relu_matmul(x, W)
    psum, pcnt = _make_sc_aggregate(N, E, D, Npad, C, rpt)(
        feat, edge_index.reshape(-1))
    return _combine(psum, pcnt, feat)


# speedup vs baseline: 17.6992x; 1.0948x over previous
import functools

import jax
import jax.numpy as jnp
from jax import lax
from jax.experimental import pallas as pl
from jax.experimental.pallas import tpu as pltpu
from jax.experimental.pallas import tpu_sc as plsc


def _mm_body(x_ref, w_ref, o_ref):
    o_ref[...] = jnp.maximum(
        jnp.dot(x_ref[...], w_ref[...], preferred_element_type=jnp.float32), 0.0)


def _relu_matmul(x, W):
    N, Din = x.shape
    Dout = W.shape[1]
    BN = 1000
    grid = (N // BN,)
    return pl.pallas_call(
        _mm_body,
        grid=grid,
        in_specs=[
            pl.BlockSpec((BN, Din), lambda i: (i, 0)),
            pl.BlockSpec((Din, Dout), lambda i: (0, 0)),
        ],
        out_specs=pl.BlockSpec((BN, Dout), lambda i: (i, 0)),
        out_shape=jax.ShapeDtypeStruct((N, Dout), jnp.float32),
    )(x, W)


def _make_sc_aggregate(N, E, D, Npad, C, rpt):
    info = plsc.get_sparse_core_info()
    NC, NS = info.num_cores, info.num_subcores
    NW = NC * NS
    nch = E // C
    base_ch = nch // NW
    extra = nch - base_ch * NW
    GK = 6
    ngr = base_ch // GK
    assert ngr * GK == base_ch
    ZR = rpt // 8

    mesh = plsc.VectorSubcoreMesh(core_axis_name="c", subcore_axis_name="s")

    @functools.partial(
        pl.kernel,
        out_type=[
            jax.ShapeDtypeStruct((NC, Npad, D), jnp.float32),
            jax.ShapeDtypeStruct((NC, Npad, 16), jnp.float32),
        ],
        mesh=mesh,
        compiler_params=pltpu.CompilerParams(use_tc_tiling_on_sc=False),
        scratch_types=[
            pltpu.VMEM_SHARED((Npad, D), jnp.float32),
            pltpu.VMEM_SHARED((Npad, 16), jnp.float32),
            pltpu.VMEM((2, GK * C), jnp.int32),
            pltpu.VMEM((2, GK * C), jnp.int32),
            pltpu.VMEM((2, C), jnp.int32),
            pltpu.VMEM((2, C, D), jnp.float32),
            pltpu.VMEM((C, 16), jnp.float32),
            pltpu.VMEM((ZR, 16), jnp.float32),
            pltpu.SemaphoreType.DMA((2,)),
            pltpu.SemaphoreType.DMA((2,)),
            pltpu.SemaphoreType.DMA((2,)),
        ],
    )
    def sc_agg(feat_hbm, edge_hbm, sum_hbm, cnt_hbm,
               acc, cnt, grow_v, gcol_v, rowm_v, rows_v, ones_v, zc_v,
               sem, gsem, ssem):
        c = lax.axis_index("c")
        s = lax.axis_index("s")
        wid = c * NS + s

        zero16 = jnp.zeros((16,), jnp.float32)
        one16 = jnp.ones((16,), jnp.float32)

        def fill_ones(i, _):
            ones_v[i, :] = one16
            return 0
        lax.fori_loop(0, C, fill_ones, 0)

        def fill_zc(i, _):
            zc_v[i, :] = zero16
            return 0
        lax.fori_loop(0, ZR, fill_zc, 0)

        def fill_z(i, _):
            rows_v[0, i // (D // 16), pl.ds((i % (D // 16)) * 16, 16)] = zero16
            return 0
        lax.fori_loop(0, C * (D // 16), fill_z, 0)

        rbase = s * rpt
        nz = rpt // C
        rem_rows = rpt - nz * C
        for k in range(nz):
            pltpu.sync_copy(rows_v.at[0], acc.at[pl.ds(rbase + k * C, C)])
        if rem_rows:
            pltpu.sync_copy(rows_v.at[0, pl.ds(0, rem_rows)],
                            acc.at[pl.ds(rbase + nz * C, rem_rows)])
        for k in range(8):
            pltpu.sync_copy(zc_v, cnt.at[pl.ds(rbase + k * ZR, ZR)])
        plsc.subcore_barrier()

        cstart = base_ch * wid

        def load_group(g):
            gb = lax.rem(g, 2)
            off = (cstart + g * GK) * C
            pltpu.async_copy(edge_hbm.at[pl.ds(off, GK * C)],
                             grow_v.at[gb], gsem.at[gb])
            pltpu.async_copy(edge_hbm.at[pl.ds(E + off, GK * C)],
                             gcol_v.at[gb], gsem.at[gb])

        def wait_group(g):
            gb = lax.rem(g, 2)
            pltpu.make_async_copy(edge_hbm.at[pl.ds(0, GK * C)],
                                  grow_v.at[gb], gsem.at[gb]).wait()
            pltpu.make_async_copy(edge_hbm.at[pl.ds(0, GK * C)],
                                  gcol_v.at[gb], gsem.at[gb]).wait()

        def stage(i):
            b = lax.rem(i, 2)
            g = i // GK
            gb = lax.rem(g, 2)
            kofs = lax.rem(i, GK) * C
            for j in range(C // 16):
                r = grow_v[gb, pl.ds(kofs + j * 16, 16)]
                cc = gcol_v[gb, pl.ds(kofs + j * 16, 16)]
                rowm_v[b, pl.ds(j * 16, 16)] = jnp.where(r == cc, N, r)
            pltpu.async_copy(feat_hbm.at[gcol_v.at[gb, pl.ds(kofs, C)]],
                             rows_v.at[b], sem.at[b])

        def wait_gather(i):
            b = lax.rem(i, 2)
            g = i // GK
            gb = lax.rem(g, 2)
            kofs = lax.rem(i, GK) * C
            pltpu.make_async_copy(
                feat_hbm.at[gcol_v.at[gb, pl.ds(kofs, C)]],
                rows_v.at[b], sem.at[b]).wait()

        def start_scatters(i):
            b = lax.rem(i, 2)
            pltpu.async_copy(rows_v.at[b], acc.at[rowm_v.at[b]],
                             ssem.at[b], add=True)
            pltpu.async_copy(ones_v, cnt.at[rowm_v.at[b]],
                             ssem.at[b], add=True)

        def wait_scatters(i):
            b = lax.rem(i, 2)
            pltpu.make_async_copy(rows_v.at[b], acc.at[pl.ds(0, C)],
                                  ssem.at[b]).wait()
            pltpu.make_async_copy(ones_v, cnt.at[pl.ds(0, C)],
                                  ssem.at[b]).wait()

        ntot = ngr * GK
        load_group(0)

        def body(i, _):
            g = i // GK
            at_group = lax.rem(i, GK) == 0

            @pl.when(i > 1)
            def _():
                wait_scatters(i - 2)

            @pl.when(jnp.logical_and(at_group, i > 0))
            def _():
                wait_gather(i - 1)

            @pl.when(at_group)
            def _():
                wait_group(g)

            @pl.when(jnp.logical_and(at_group, g + 1 < ngr))
            def _():
                load_group(g + 1)

            stage(i)

            @pl.when(jnp.logical_and(jnp.logical_not(at_group), i > 0))
            def _():
                wait_gather(i - 1)

            @pl.when(i > 0)
            def _():
                start_scatters(i - 1)
            return 0
        lax.fori_loop(0, ntot, body, 0)
        wait_gather(ntot - 1)
        start_scatters(ntot - 1)
        wait_scatters(ntot - 2)
        wait_scatters(ntot - 1)

        @pl.when(wid < extra)
        def _():
            toff = (NW * base_ch + wid) * C
            pltpu.sync_copy(edge_hbm.at[pl.ds(toff, C)],
                            grow_v.at[0, pl.ds(0, C)])
            pltpu.sync_copy(edge_hbm.at[pl.ds(E + toff, C)],
                            gcol_v.at[0, pl.ds(0, C)])
            for j in range(C // 16):
                r = grow_v[0, pl.ds(j * 16, 16)]
                cc = gcol_v[0, pl.ds(j * 16, 16)]
                rowm_v[0, pl.ds(j * 16, 16)] = jnp.where(r == cc, N, r)
            pltpu.async_copy(feat_hbm.at[gcol_v.at[0, pl.ds(0, C)]],
                             rows_v.at[0], sem.at[0])
            pltpu.make_async_copy(
                feat_hbm.at[gcol_v.at[0, pl.ds(0, C)]],
                rows_v.at[0], sem.at[0]).wait()
            pltpu.sync_copy(rows_v.at[0], acc.at[rowm_v.at[0]], add=True)
            pltpu.sync_copy(ones_v, cnt.at[rowm_v.at[0]], add=True)

        plsc.subcore_barrier()
        pltpu.sync_copy(acc.at[pl.ds(rbase, rpt)], sum_hbm.at[c, pl.ds(rbase, rpt)])
        pltpu.sync_copy(cnt.at[pl.ds(rbase, rpt)], cnt_hbm.at[c, pl.ds(rbase, rpt)])

    return sc_agg


def _div_body(ps_ref, pc_ref, feat_ref, o_ref):
    total = ps_ref[0] + ps_ref[1] + feat_ref[...]
    den = pc_ref[0, :, 0:1] + pc_ref[1, :, 0:1] + 1.0
    o_ref[...] = total / den


def _combine(psum, pcnt, feat):
    N, D = feat.shape
    BN = 400
    grid = (N // BN,)
    return pl.pallas_call(
        _div_body,
        grid=grid,
        in_specs=[
            pl.BlockSpec((2, BN, D), lambda i: (0, i, 0)),
            pl.BlockSpec((2, BN, 16), lambda i: (0, i, 0)),
            pl.BlockSpec((BN, D), lambda i: (i, 0)),
        ],
        out_specs=pl.BlockSpec((BN, D), lambda i: (i, 0)),
        out_shape=jax.ShapeDtypeStruct((N, D), jnp.float32),
    )(psum, pcnt, feat)


def kernel(x, edge_index, W):
    N, _ = x.shape
    D = W.shape[1]
    E = edge_index.shape[1]

    info = plsc.get_sparse_core_info()
    NS = info.num_subcores
    C = 128
    assert E % C == 0
    rpt = -(-(N + 1) // NS)
    rpt = -(-rpt // 8) * 8
    Npad = rpt * NS

    feat = _relu_matmul(x, W)
    psum, pcnt = _make_sc_aggregate(N, E, D, Npad, C, rpt)(
        feat, edge_index.reshape(-1))
    return _combine(psum, pcnt, feat)
